# baseline (speedup 1.0000x reference)
import functools

import jax
import jax.numpy as jnp
from jax.experimental import pallas as pl
from jax.experimental.pallas import tpu as pltpu

_BF = jnp.bfloat16
_F32 = jnp.float32
_VMEM_LIMIT = 100 * 1024 * 1024


def _const2(i):
  return (0, 0)


def _strided_hw(a, di, dj, ho, wo):
  g, hp, wp, c = a.shape
  a6 = a.reshape(g, hp // 2, 2, wp // 2, 2, c)
  return a6[:, di // 2:di // 2 + ho, di % 2, dj // 2:dj // 2 + wo, dj % 2, :]


def _bottleneck_body(x_ref, w1_ref, b1_ref, w2_ref, b2_ref, w3_ref, b3_ref,
                     *rest, g, h, w, stride, has_ds):
  if has_ds:
    wd_ref, bd_ref, o_ref, pad_ref, col_ref = rest
  else:
    o_ref, pad_ref, col_ref = rest
  ho, wo = h // stride, w // stride
  width = w1_ref.shape[1]
  cin = x_ref.shape[1]

  x = x_ref[...]
  h1 = jnp.dot(x, w1_ref[...], preferred_element_type=_F32) + b1_ref[...]
  h1 = jnp.maximum(h1, 0.0).astype(_BF)

  z = jnp.zeros((g, 1, w + 2, width), _BF)
  pad_ref[:, 0:1, :, :] = z
  pad_ref[:, h + 1:h + 2, :, :] = z
  zc = jnp.zeros((g, h + 2, 1, width), _BF)
  pad_ref[:, :, 0:1, :] = zc
  pad_ref[:, :, w + 1:w + 2, :] = zc
  pad_ref[:, 1:h + 1, 1:w + 1, :] = h1.reshape(g, h, w, width)

  m = pad_ref[...]
  for t in range(9):
    di, dj = t // 3, t % 3
    if stride == 1:
      patch = m[:, di:di + h, dj:dj + w, :]
    else:
      patch = _strided_hw(m, di, dj, ho, wo)
    col_ref[:, t * width:(t + 1) * width] = patch.reshape(g * ho * wo, width)

  h2 = jnp.dot(col_ref[...], w2_ref[...], preferred_element_type=_F32)
  h2 = jnp.maximum(h2 + b2_ref[...], 0.0).astype(_BF)

  if has_ds:
    if stride == 1:
      xs = x
    else:
      xs = _strided_hw(x.reshape(g, h, w, cin), 0, 0, ho, wo)
      xs = xs.reshape(g * ho * wo, cin)
    idt = jnp.dot(xs, wd_ref[...], preferred_element_type=_F32) + bd_ref[...]
  else:
    idt = x.astype(_F32)

  y = jnp.dot(h2, w3_ref[...], preferred_element_type=_F32) + b3_ref[...] + idt
  o_ref[...] = jnp.maximum(y, 0.0).astype(_BF)


def _bottleneck(x, p1, p2, p3, pd, n, h, w, cin, width, cout, stride, g):
  ho, wo = h // stride, w // stride
  w1 = p1[0].reshape(cin, width).astype(_BF)
  b1 = p1[1].reshape(1, width).astype(_F32)
  w2 = p2[0].reshape(9 * width, width).astype(_BF)
  b2 = p2[1].reshape(1, width).astype(_F32)
  w3 = p3[0].reshape(width, cout).astype(_BF)
  b3 = p3[1].reshape(1, cout).astype(_F32)
  has_ds = pd is not None

  ins = [x, w1, b1, w2, b2, w3, b3]
  in_specs = [
      pl.BlockSpec((g * h * w, cin), lambda i: (i, 0)),
      pl.BlockSpec((cin, width), _const2),
      pl.BlockSpec((1, width), _const2),
      pl.BlockSpec((9 * width, width), _const2),
      pl.BlockSpec((1, width), _const2),
      pl.BlockSpec((width, cout), _const2),
      pl.BlockSpec((1, cout), _const2),
  ]
  if has_ds:
    ins += [pd[0].reshape(cin, cout).astype(_BF),
            pd[1].reshape(1, cout).astype(_F32)]
    in_specs += [pl.BlockSpec((cin, cout), _const2),
                 pl.BlockSpec((1, cout), _const2)]

  body = functools.partial(_bottleneck_body, g=g, h=h, w=w, stride=stride,
                           has_ds=has_ds)
  flops_g = 2 * g * (h * w * cin * width + ho * wo * 9 * width * width
                     + ho * wo * width * cout
                     + (ho * wo * cin * cout if has_ds else 0))
  return pl.pallas_call(
      body,
      out_shape=jax.ShapeDtypeStruct((n * ho * wo, cout), _BF),
      grid=(n // g,),
      in_specs=in_specs,
      out_specs=pl.BlockSpec((g * ho * wo, cout), lambda i: (i, 0)),
      scratch_shapes=[pltpu.VMEM((g, h + 2, w + 2, width), _BF),
                      pltpu.VMEM((g * ho * wo, 9 * width), _BF)],
      compiler_params=pltpu.CompilerParams(
          dimension_semantics=("parallel",),
          vmem_limit_bytes=_VMEM_LIMIT),
      cost_estimate=pl.CostEstimate(
          flops=flops_g * (n // g), transcendentals=0,
          bytes_accessed=2 * (n * h * w * cin + n * ho * wo * cout)),
  )(*ins)


def _stem_body(x_ref, w_ref, b_ref, m_ref, o_ref, col_ref):
  m = x_ref[0]
  for bb in range(4):
    col_ref[:, bb * 48:(bb + 1) * 48] = m[:, bb:bb + 112, :].reshape(12544, 48)
  y = jnp.dot(col_ref[...], w_ref[...], preferred_element_type=_F32)
  y = jnp.maximum(y + b_ref[...], 0.0).astype(_BF)
  y4 = y.reshape(6272, 2, 64)
  e, o = y4[:, 0], y4[:, 1]
  a = jnp.maximum(e, o)
  po = jnp.concatenate([jnp.zeros((1, 64), _BF), o[:6271]], axis=0)
  h = jnp.maximum(a, po * m_ref[...])
  z = h.reshape(56, 2, 56, 64)
  ev, ov = z[:, 0], z[:, 1]
  a2 = jnp.maximum(ev, ov)
  pv = jnp.concatenate([jnp.zeros((1, 56, 64), _BF), ov[:55]], axis=0)
  o_ref[...] = jnp.maximum(a2, pv).reshape(3136, 64)


def _stem(x_nchw, w, b):
  n = x_nchw.shape[0]
  xb = x_nchw.astype(_BF)
  xp = jnp.pad(xb, ((0, 0), (0, 0), (3, 3), (3, 3)))
  xs = (xp.reshape(n, 3, 115, 2, 115, 2)
        .transpose(0, 2, 4, 3, 5, 1).reshape(n, 115, 115, 12))
  colA = jnp.concatenate([xs[:, a:a + 112, :, :] for a in range(4)], axis=-1)
  w8 = jnp.pad(w.astype(_BF), ((0, 1), (0, 1), (0, 0), (0, 0)))
  wm = (w8.reshape(4, 2, 4, 2, 3, 64)
        .transpose(2, 0, 1, 3, 4, 5).reshape(192, 64))
  bm = b.reshape(1, 64).astype(_F32)
  msk = (jnp.arange(6272, dtype=jnp.int32) % 56 != 0).astype(_BF).reshape(6272, 1)
  return pl.pallas_call(
      _stem_body,
      out_shape=jax.ShapeDtypeStruct((n * 56 * 56, 64), _BF),
      grid=(n,),
      in_specs=[pl.BlockSpec((1, 112, 115, 48), lambda i: (i, 0, 0, 0)),
                pl.BlockSpec((192, 64), _const2),
                pl.BlockSpec((1, 64), _const2),
                pl.BlockSpec((6272, 1), _const2)],
      out_specs=pl.BlockSpec((3136, 64), lambda i: (i, 0)),
      scratch_shapes=[pltpu.VMEM((12544, 192), _BF)],
      compiler_params=pltpu.CompilerParams(
          dimension_semantics=("parallel",),
          vmem_limit_bytes=_VMEM_LIMIT),
  )(colA, wm, bm, msk)


def _head_body(x_ref, w_ref, b_ref, o_ref):
  m = x_ref.shape[0] // 49
  x = x_ref[...].astype(_F32).reshape(m, 49, 2048)
  feat = jnp.mean(x, axis=1).astype(_BF)
  o_ref[...] = jnp.dot(feat, w_ref[...], preferred_element_type=_F32) + b_ref[...]


def _head(x, fc_w, fc_b):
  n = x.shape[0] // 49
  classes = fc_w.shape[1]
  return pl.pallas_call(
      _head_body,
      out_shape=jax.ShapeDtypeStruct((n, classes), _F32),
      grid=(1,),
      in_specs=[pl.BlockSpec((n * 49, 2048), _const2),
                pl.BlockSpec((2048, classes), _const2),
                pl.BlockSpec((1, classes), _const2)],
      out_specs=pl.BlockSpec((n, classes), _const2),
      compiler_params=pltpu.CompilerParams(
          dimension_semantics=("arbitrary",),
          vmem_limit_bytes=_VMEM_LIMIT),
  )(x.astype(_BF), fc_w.astype(_BF), fc_b.reshape(1, classes).astype(_F32))


_STAGE_GEOM = {
    (0, 0): (56, 64, 64, 256, 1, 1),
    (0, 1): (56, 256, 64, 256, 1, 1),
    (1, 0): (56, 256, 128, 512, 2, 1),
    (1, 1): (28, 512, 128, 512, 1, 1),
    (2, 0): (28, 512, 256, 1024, 2, 2),
    (2, 1): (14, 1024, 256, 1024, 1, 4),
    (3, 0): (14, 1024, 512, 2048, 2, 8),
    (3, 1): (7, 2048, 512, 2048, 1, 8),
}


def kernel(stem_conv1_w, stem_conv1_bias, L0_B0_conv1_w, L0_B0_conv1_bias, L0_B0_conv2_w, L0_B0_conv2_bias, L0_B0_conv3_w, L0_B0_conv3_bias, L0_B0_downsample_w, L0_B0_downsample_bias, L0_B1_conv1_w, L0_B1_conv1_bias, L0_B1_conv2_w, L0_B1_conv2_bias, L0_B1_conv3_w, L0_B1_conv3_bias, L0_B2_conv1_w, L0_B2_conv1_bias, L0_B2_conv2_w, L0_B2_conv2_bias, L0_B2_conv3_w, L0_B2_conv3_bias, L1_B0_conv1_w, L1_B0_conv1_bias, L1_B0_conv2_w, L1_B0_conv2_bias, L1_B0_conv3_w, L1_B0_conv3_bias, L1_B0_downsample_w, L1_B0_downsample_bias, L1_B1_conv1_w, L1_B1_conv1_bias, L1_B1_conv2_w, L1_B1_conv2_bias, L1_B1_conv3_w, L1_B1_conv3_bias, L1_B2_conv1_w, L1_B2_conv1_bias, L1_B2_conv2_w, L1_B2_conv2_bias, L1_B2_conv3_w, L1_B2_conv3_bias, L1_B3_conv1_w, L1_B3_conv1_bias, L1_B3_conv2_w, L1_B3_conv2_bias, L1_B3_conv3_w, L1_B3_conv3_bias, L2_B0_conv1_w, L2_B0_conv1_bias, L2_B0_conv2_w, L2_B0_conv2_bias, L2_B0_conv3_w, L2_B0_conv3_bias, L2_B0_downsample_w, L2_B0_downsample_bias, L2_B1_conv1_w, L2_B1_conv1_bias, L2_B1_conv2_w, L2_B1_conv2_bias, L2_B1_conv3_w, L2_B1_conv3_bias, L2_B2_conv1_w, L2_B2_conv1_bias, L2_B2_conv2_w, L2_B2_conv2_bias, L2_B2_conv3_w, L2_B2_conv3_bias, L2_B3_conv1_w, L2_B3_conv1_bias, L2_B3_conv2_w, L2_B3_conv2_bias, L2_B3_conv3_w, L2_B3_conv3_bias, L2_B4_conv1_w, L2_B4_conv1_bias, L2_B4_conv2_w, L2_B4_conv2_bias, L2_B4_conv3_w, L2_B4_conv3_bias, L2_B5_conv1_w, L2_B5_conv1_bias, L2_B5_conv2_w, L2_B5_conv2_bias, L2_B5_conv3_w, L2_B5_conv3_bias, L3_B0_conv1_w, L3_B0_conv1_bias, L3_B0_conv2_w, L3_B0_conv2_bias, L3_B0_conv3_w, L3_B0_conv3_bias, L3_B0_downsample_w, L3_B0_downsample_bias, L3_B1_conv1_w, L3_B1_conv1_bias, L3_B1_conv2_w, L3_B1_conv2_bias, L3_B1_conv3_w, L3_B1_conv3_bias, L3_B2_conv1_w, L3_B2_conv1_bias, L3_B2_conv2_w, L3_B2_conv2_bias, L3_B2_conv3_w, L3_B2_conv3_bias, fc_w, fc_b, x):
  n = x.shape[0]
  args = locals()
  a = _stem(x, stem_conv1_w, stem_conv1_bias)

  nblocks = (3, 4, 6, 3)
  for li in range(4):
    for bi in range(nblocks[li]):
      pre = f"L{li}_B{bi}_"
      p1 = (args[pre + "conv1_w"], args[pre + "conv1_bias"])
      p2 = (args[pre + "conv2_w"], args[pre + "conv2_bias"])
      p3 = (args[pre + "conv3_w"], args[pre + "conv3_bias"])
      pd = None
      if bi == 0:
        pd = (args[pre + "downsample_w"], args[pre + "downsample_bias"])
      h, cin, width, cout, stride, g = _STAGE_GEOM[(li, min(bi, 1))]
      g = min(g, n)
      a = _bottleneck(a, p1, p2, p3, pd, n, h, h, cin, width, cout, stride, g)

  return _head(a, fc_w, fc_b)


# speedup vs baseline: 1.0366x; 1.0366x over previous
import functools

import jax
import jax.numpy as jnp
from jax.experimental import pallas as pl
from jax.experimental.pallas import tpu as pltpu

_BF = jnp.bfloat16
_F32 = jnp.float32
_VMEM_LIMIT = 100 * 1024 * 1024


def _const2(i):
  return (0, 0)


def _strided_hw(a, di, dj, ho, wo):
  g, hp, wp, c = a.shape
  a6 = a.reshape(g, hp // 2, 2, wp // 2, 2, c)
  return a6[:, di // 2:di // 2 + ho, di % 2, dj // 2:dj // 2 + wo, dj % 2, :]


def _bottleneck_body(x_ref, w1_ref, b1_ref, w2_ref, b2_ref, w3_ref, b3_ref,
                     *rest, g, h, w, stride, has_ds):
  if has_ds:
    wd_ref, bd_ref, o_ref, pad_ref, col_ref = rest
  else:
    o_ref, pad_ref, col_ref = rest
  ho, wo = h // stride, w // stride
  width = w1_ref.shape[1]
  cin = x_ref.shape[1]

  x = x_ref[...]
  h1 = jnp.dot(x, w1_ref[...], preferred_element_type=_F32) + b1_ref[...]
  h1 = jnp.maximum(h1, 0.0).astype(_BF)

  z = jnp.zeros((g, 1, w + 2, width), _BF)
  pad_ref[:, 0:1, :, :] = z
  pad_ref[:, h + 1:h + 2, :, :] = z
  zc = jnp.zeros((g, h + 2, 1, width), _BF)
  pad_ref[:, :, 0:1, :] = zc
  pad_ref[:, :, w + 1:w + 2, :] = zc
  pad_ref[:, 1:h + 1, 1:w + 1, :] = h1.reshape(g, h, w, width)

  m = pad_ref[...]
  for t in range(9):
    di, dj = t // 3, t % 3
    if stride == 1:
      patch = m[:, di:di + h, dj:dj + w, :]
    else:
      patch = _strided_hw(m, di, dj, ho, wo)
    col_ref[:, t * width:(t + 1) * width] = patch.reshape(g * ho * wo, width)

  h2 = jnp.dot(col_ref[...], w2_ref[...], preferred_element_type=_F32)
  h2 = jnp.maximum(h2 + b2_ref[...], 0.0).astype(_BF)

  if has_ds:
    if stride == 1:
      xs = x
    else:
      xs = _strided_hw(x.reshape(g, h, w, cin), 0, 0, ho, wo)
      xs = xs.reshape(g * ho * wo, cin)
    idt = jnp.dot(xs, wd_ref[...], preferred_element_type=_F32) + bd_ref[...]
  else:
    idt = x.astype(_F32)

  y = jnp.dot(h2, w3_ref[...], preferred_element_type=_F32) + b3_ref[...] + idt
  o_ref[...] = jnp.maximum(y, 0.0).astype(_BF)


def _bottleneck(x, p1, p2, p3, pd, n, h, w, cin, width, cout, stride, g):
  ho, wo = h // stride, w // stride
  w1 = p1[0].reshape(cin, width).astype(_BF)
  b1 = p1[1].reshape(1, width).astype(_F32)
  w2 = p2[0].reshape(9 * width, width).astype(_BF)
  b2 = p2[1].reshape(1, width).astype(_F32)
  w3 = p3[0].reshape(width, cout).astype(_BF)
  b3 = p3[1].reshape(1, cout).astype(_F32)
  has_ds = pd is not None

  ins = [x, w1, b1, w2, b2, w3, b3]
  in_specs = [
      pl.BlockSpec((g * h * w, cin), lambda i: (i, 0)),
      pl.BlockSpec((cin, width), _const2),
      pl.BlockSpec((1, width), _const2),
      pl.BlockSpec((9 * width, width), _const2),
      pl.BlockSpec((1, width), _const2),
      pl.BlockSpec((width, cout), _const2),
      pl.BlockSpec((1, cout), _const2),
  ]
  if has_ds:
    ins += [pd[0].reshape(cin, cout).astype(_BF),
            pd[1].reshape(1, cout).astype(_F32)]
    in_specs += [pl.BlockSpec((cin, cout), _const2),
                 pl.BlockSpec((1, cout), _const2)]

  body = functools.partial(_bottleneck_body, g=g, h=h, w=w, stride=stride,
                           has_ds=has_ds)
  flops_g = 2 * g * (h * w * cin * width + ho * wo * 9 * width * width
                     + ho * wo * width * cout
                     + (ho * wo * cin * cout if has_ds else 0))
  return pl.pallas_call(
      body,
      out_shape=jax.ShapeDtypeStruct((n * ho * wo, cout), _BF),
      grid=(n // g,),
      in_specs=in_specs,
      out_specs=pl.BlockSpec((g * ho * wo, cout), lambda i: (i, 0)),
      scratch_shapes=[pltpu.VMEM((g, h + 2, w + 2, width), _BF),
                      pltpu.VMEM((g * ho * wo, 9 * width), _BF)],
      compiler_params=pltpu.CompilerParams(
          dimension_semantics=("parallel",),
          vmem_limit_bytes=_VMEM_LIMIT),
      cost_estimate=pl.CostEstimate(
          flops=flops_g * (n // g), transcendentals=0,
          bytes_accessed=2 * (n * h * w * cin + n * ho * wo * cout)),
  )(*ins)


def _stem_body(x_ref, w_ref, b_ref, o_ref, p_ref, col_ref):
  m = x_ref[0]
  for bb in range(4):
    col_ref[:, bb * 48:(bb + 1) * 48] = m[:, bb:bb + 112, :].reshape(12544, 48)
  y = jnp.dot(col_ref[...], w_ref[...], preferred_element_type=_F32)
  y = jnp.maximum(y + b_ref[...], 0.0).astype(_BF)
  z = jnp.zeros((1, 114, 64), _BF)
  p_ref[0:1, :, :] = z
  p_ref[113:114, :, :] = z
  zc = jnp.zeros((114, 1, 64), _BF)
  p_ref[:, 0:1, :] = zc
  p_ref[:, 113:114, :] = zc
  p_ref[1:113, 1:113, :] = y.reshape(112, 112, 64)
  m6 = p_ref[...].reshape(57, 2, 57, 2, 64)
  r = None
  for di in range(3):
    for dj in range(3):
      s = m6[di // 2:di // 2 + 56, di % 2, dj // 2:dj // 2 + 56, dj % 2, :]
      r = s if r is None else jnp.maximum(r, s)
  o_ref[...] = r.reshape(3136, 64)


def _stem(x_nchw, w, b):
  n = x_nchw.shape[0]
  xb = x_nchw.astype(_BF)
  xp = jnp.pad(xb, ((0, 0), (0, 0), (3, 3), (3, 3)))
  xs = (xp.reshape(n, 3, 115, 2, 115, 2)
        .transpose(0, 2, 4, 3, 5, 1).reshape(n, 115, 115, 12))
  colA = jnp.concatenate([xs[:, a:a + 112, :, :] for a in range(4)], axis=-1)
  w8 = jnp.pad(w.astype(_BF), ((0, 1), (0, 1), (0, 0), (0, 0)))
  wm = (w8.reshape(4, 2, 4, 2, 3, 64)
        .transpose(2, 0, 1, 3, 4, 5).reshape(192, 64))
  bm = b.reshape(1, 64).astype(_F32)
  return pl.pallas_call(
      _stem_body,
      out_shape=jax.ShapeDtypeStruct((n * 56 * 56, 64), _BF),
      grid=(n,),
      in_specs=[pl.BlockSpec((1, 112, 115, 48), lambda i: (i, 0, 0, 0)),
                pl.BlockSpec((192, 64), _const2),
                pl.BlockSpec((1, 64), _const2)],
      out_specs=pl.BlockSpec((3136, 64), lambda i: (i, 0)),
      scratch_shapes=[pltpu.VMEM((114, 114, 64), _BF),
                      pltpu.VMEM((12544, 192), _BF)],
      compiler_params=pltpu.CompilerParams(
          dimension_semantics=("parallel",),
          vmem_limit_bytes=_VMEM_LIMIT),
  )(colA, wm, bm)


def _head_body(x_ref, w_ref, b_ref, o_ref):
  m = x_ref.shape[0] // 49
  x = x_ref[...].astype(_F32).reshape(m, 49, 2048)
  feat = jnp.mean(x, axis=1).astype(_BF)
  o_ref[...] = jnp.dot(feat, w_ref[...], preferred_element_type=_F32) + b_ref[...]


def _head(x, fc_w, fc_b):
  n = x.shape[0] // 49
  classes = fc_w.shape[1]
  return pl.pallas_call(
      _head_body,
      out_shape=jax.ShapeDtypeStruct((n, classes), _F32),
      grid=(1,),
      in_specs=[pl.BlockSpec((n * 49, 2048), _const2),
                pl.BlockSpec((2048, classes), _const2),
                pl.BlockSpec((1, classes), _const2)],
      out_specs=pl.BlockSpec((n, classes), _const2),
      compiler_params=pltpu.CompilerParams(
          dimension_semantics=("arbitrary",),
          vmem_limit_bytes=_VMEM_LIMIT),
  )(x.astype(_BF), fc_w.astype(_BF), fc_b.reshape(1, classes).astype(_F32))


_STAGE_GEOM = {
    (0, 0): (56, 64, 64, 256, 1, 1),
    (0, 1): (56, 256, 64, 256, 1, 1),
    (1, 0): (56, 256, 128, 512, 2, 1),
    (1, 1): (28, 512, 128, 512, 1, 1),
    (2, 0): (28, 512, 256, 1024, 2, 2),
    (2, 1): (14, 1024, 256, 1024, 1, 4),
    (3, 0): (14, 1024, 512, 2048, 2, 8),
    (3, 1): (7, 2048, 512, 2048, 1, 8),
}


def kernel(stem_conv1_w, stem_conv1_bias, L0_B0_conv1_w, L0_B0_conv1_bias, L0_B0_conv2_w, L0_B0_conv2_bias, L0_B0_conv3_w, L0_B0_conv3_bias, L0_B0_downsample_w, L0_B0_downsample_bias, L0_B1_conv1_w, L0_B1_conv1_bias, L0_B1_conv2_w, L0_B1_conv2_bias, L0_B1_conv3_w, L0_B1_conv3_bias, L0_B2_conv1_w, L0_B2_conv1_bias, L0_B2_conv2_w, L0_B2_conv2_bias, L0_B2_conv3_w, L0_B2_conv3_bias, L1_B0_conv1_w, L1_B0_conv1_bias, L1_B0_conv2_w, L1_B0_conv2_bias, L1_B0_conv3_w, L1_B0_conv3_bias, L1_B0_downsample_w, L1_B0_downsample_bias, L1_B1_conv1_w, L1_B1_conv1_bias, L1_B1_conv2_w, L1_B1_conv2_bias, L1_B1_conv3_w, L1_B1_conv3_bias, L1_B2_conv1_w, L1_B2_conv1_bias, L1_B2_conv2_w, L1_B2_conv2_bias, L1_B2_conv3_w, L1_B2_conv3_bias, L1_B3_conv1_w, L1_B3_conv1_bias, L1_B3_conv2_w, L1_B3_conv2_bias, L1_B3_conv3_w, L1_B3_conv3_bias, L2_B0_conv1_w, L2_B0_conv1_bias, L2_B0_conv2_w, L2_B0_conv2_bias, L2_B0_conv3_w, L2_B0_conv3_bias, L2_B0_downsample_w, L2_B0_downsample_bias, L2_B1_conv1_w, L2_B1_conv1_bias, L2_B1_conv2_w, L2_B1_conv2_bias, L2_B1_conv3_w, L2_B1_conv3_bias, L2_B2_conv1_w, L2_B2_conv1_bias, L2_B2_conv2_w, L2_B2_conv2_bias, L2_B2_conv3_w, L2_B2_conv3_bias, L2_B3_conv1_w, L2_B3_conv1_bias, L2_B3_conv2_w, L2_B3_conv2_bias, L2_B3_conv3_w, L2_B3_conv3_bias, L2_B4_conv1_w, L2_B4_conv1_bias, L2_B4_conv2_w, L2_B4_conv2_bias, L2_B4_conv3_w, L2_B4_conv3_bias, L2_B5_conv1_w, L2_B5_conv1_bias, L2_B5_conv2_w, L2_B5_conv2_bias, L2_B5_conv3_w, L2_B5_conv3_bias, L3_B0_conv1_w, L3_B0_conv1_bias, L3_B0_conv2_w, L3_B0_conv2_bias, L3_B0_conv3_w, L3_B0_conv3_bias, L3_B0_downsample_w, L3_B0_downsample_bias, L3_B1_conv1_w, L3_B1_conv1_bias, L3_B1_conv2_w, L3_B1_conv2_bias, L3_B1_conv3_w, L3_B1_conv3_bias, L3_B2_conv1_w, L3_B2_conv1_bias, L3_B2_conv2_w, L3_B2_conv2_bias, L3_B2_conv3_w, L3_B2_conv3_bias, fc_w, fc_b, x):
  n = x.shape[0]
  args = locals()
  a = _stem(x, stem_conv1_w, stem_conv1_bias)

  nblocks = (3, 4, 6, 3)
  for li in range(4):
    for bi in range(nblocks[li]):
      pre = f"L{li}_B{bi}_"
      p1 = (args[pre + "conv1_w"], args[pre + "conv1_bias"])
      p2 = (args[pre + "conv2_w"], args[pre + "conv2_bias"])
      p3 = (args[pre + "conv3_w"], args[pre + "conv3_bias"])
      pd = None
      if bi == 0:
        pd = (args[pre + "downsample_w"], args[pre + "downsample_bias"])
      h, cin, width, cout, stride, g = _STAGE_GEOM[(li, min(bi, 1))]
      g = min(g, n)
      a = _bottleneck(a, p1, p2, p3, pd, n, h, h, cin, width, cout, stride, g)

  return _head(a, fc_w, fc_b)


# speedup vs baseline: 1.0428x; 1.0060x over previous
import functools

import jax
import jax.numpy as jnp
from jax.experimental import pallas as pl
from jax.experimental.pallas import tpu as pltpu

_BF = jnp.bfloat16
_F32 = jnp.float32
_VMEM_LIMIT = 100 * 1024 * 1024


def _const2(i):
  return (0, 0)


def _strided_hw(a, di, dj, ho, wo):
  g, hp, wp, c = a.shape
  a6 = a.reshape(g, hp // 2, 2, wp // 2, 2, c)
  return a6[:, di // 2:di // 2 + ho, di % 2, dj // 2:dj // 2 + wo, dj % 2, :]


def _bottleneck_body(x_ref, w1_ref, b1_ref, w2_ref, b2_ref, w3_ref, b3_ref,
                     *rest, g, h, w, stride, has_ds):
  if has_ds:
    wd_ref, bd_ref, o_ref, pad_ref, col_ref = rest
  else:
    o_ref, pad_ref, col_ref = rest
  ho, wo = h // stride, w // stride
  width = w1_ref.shape[1]
  cin = x_ref.shape[1]

  x = x_ref[...]
  h1 = jnp.dot(x, w1_ref[...], preferred_element_type=_F32) + b1_ref[...]
  h1 = jnp.maximum(h1, 0.0).astype(_BF)

  z = jnp.zeros((g, 1, w + 2, width), _BF)
  pad_ref[:, 0:1, :, :] = z
  pad_ref[:, h + 1:h + 2, :, :] = z
  zc = jnp.zeros((g, h + 2, 1, width), _BF)
  pad_ref[:, :, 0:1, :] = zc
  pad_ref[:, :, w + 1:w + 2, :] = zc
  pad_ref[:, 1:h + 1, 1:w + 1, :] = h1.reshape(g, h, w, width)

  m = pad_ref[...]
  for t in range(9):
    di, dj = t // 3, t % 3
    if stride == 1:
      patch = m[:, di:di + h, dj:dj + w, :]
    else:
      patch = _strided_hw(m, di, dj, ho, wo)
    col_ref[:, t * width:(t + 1) * width] = patch.reshape(g * ho * wo, width)

  h2 = jnp.dot(col_ref[...], w2_ref[...], preferred_element_type=_F32)
  h2 = jnp.maximum(h2 + b2_ref[...], 0.0).astype(_BF)

  if has_ds:
    if stride == 1:
      xs = x
    else:
      xs = _strided_hw(x.reshape(g, h, w, cin), 0, 0, ho, wo)
      xs = xs.reshape(g * ho * wo, cin)
    idt = jnp.dot(xs, wd_ref[...], preferred_element_type=_F32) + bd_ref[...]
  else:
    idt = x.astype(_F32)

  y = jnp.dot(h2, w3_ref[...], preferred_element_type=_F32) + b3_ref[...] + idt
  o_ref[...] = jnp.maximum(y, 0.0).astype(_BF)


def _bottleneck(x, p1, p2, p3, pd, n, h, w, cin, width, cout, stride, g):
  ho, wo = h // stride, w // stride
  w1 = p1[0].reshape(cin, width).astype(_BF)
  b1 = p1[1].reshape(1, width).astype(_F32)
  w2 = p2[0].reshape(9 * width, width).astype(_BF)
  b2 = p2[1].reshape(1, width).astype(_F32)
  w3 = p3[0].reshape(width, cout).astype(_BF)
  b3 = p3[1].reshape(1, cout).astype(_F32)
  has_ds = pd is not None

  ins = [x, w1, b1, w2, b2, w3, b3]
  in_specs = [
      pl.BlockSpec((g * h * w, cin), lambda i: (i, 0)),
      pl.BlockSpec((cin, width), _const2),
      pl.BlockSpec((1, width), _const2),
      pl.BlockSpec((9 * width, width), _const2),
      pl.BlockSpec((1, width), _const2),
      pl.BlockSpec((width, cout), _const2),
      pl.BlockSpec((1, cout), _const2),
  ]
  if has_ds:
    ins += [pd[0].reshape(cin, cout).astype(_BF),
            pd[1].reshape(1, cout).astype(_F32)]
    in_specs += [pl.BlockSpec((cin, cout), _const2),
                 pl.BlockSpec((1, cout), _const2)]

  body = functools.partial(_bottleneck_body, g=g, h=h, w=w, stride=stride,
                           has_ds=has_ds)
  flops_g = 2 * g * (h * w * cin * width + ho * wo * 9 * width * width
                     + ho * wo * width * cout
                     + (ho * wo * cin * cout if has_ds else 0))
  return pl.pallas_call(
      body,
      out_shape=jax.ShapeDtypeStruct((n * ho * wo, cout), _BF),
      grid=(n // g,),
      in_specs=in_specs,
      out_specs=pl.BlockSpec((g * ho * wo, cout), lambda i: (i, 0)),
      scratch_shapes=[pltpu.VMEM((g, h + 2, w + 2, width), _BF),
                      pltpu.VMEM((g * ho * wo, 9 * width), _BF)],
      compiler_params=pltpu.CompilerParams(
          dimension_semantics=("parallel",),
          vmem_limit_bytes=_VMEM_LIMIT),
      cost_estimate=pl.CostEstimate(
          flops=flops_g * (n // g), transcendentals=0,
          bytes_accessed=2 * (n * h * w * cin + n * ho * wo * cout)),
  )(*ins)


def _stem_body(x_ref, w_ref, b_ref, o_ref, p_ref, col_ref):
  m = x_ref[0]
  for bb in range(4):
    col_ref[:, bb * 48:(bb + 1) * 48] = m[:, bb:bb + 112, :].reshape(12544, 48)
  y = jnp.dot(col_ref[...], w_ref[...], preferred_element_type=_F32)
  y = jnp.maximum(y + b_ref[...], 0.0).astype(_BF)
  z = jnp.zeros((1, 114, 64), _BF)
  p_ref[0:1, :, :] = z
  p_ref[113:114, :, :] = z
  zc = jnp.zeros((114, 1, 64), _BF)
  p_ref[:, 0:1, :] = zc
  p_ref[:, 113:114, :] = zc
  p_ref[1:113, 1:113, :] = y.reshape(112, 112, 64)
  m6 = p_ref[...].reshape(57, 2, 57, 2, 64)
  r = None
  for di in range(3):
    for dj in range(3):
      s = m6[di // 2:di // 2 + 56, di % 2, dj // 2:dj // 2 + 56, dj % 2, :]
      r = s if r is None else jnp.maximum(r, s)
  o_ref[...] = r.reshape(3136, 64)


def _stem(x_nchw, w, b):
  n = x_nchw.shape[0]
  xb = x_nchw.astype(_BF)
  xp = jnp.pad(xb, ((0, 0), (0, 0), (3, 3), (3, 3)))
  xs = (xp.reshape(n, 3, 115, 2, 115, 2)
        .transpose(0, 2, 4, 3, 5, 1).reshape(n, 115, 115, 12))
  colA = jnp.concatenate([xs[:, a:a + 112, :, :] for a in range(4)], axis=-1)
  w8 = jnp.pad(w.astype(_BF), ((0, 1), (0, 1), (0, 0), (0, 0)))
  wm = (w8.reshape(4, 2, 4, 2, 3, 64)
        .transpose(2, 0, 1, 3, 4, 5).reshape(192, 64))
  bm = b.reshape(1, 64).astype(_F32)
  return pl.pallas_call(
      _stem_body,
      out_shape=jax.ShapeDtypeStruct((n * 56 * 56, 64), _BF),
      grid=(n,),
      in_specs=[pl.BlockSpec((1, 112, 115, 48), lambda i: (i, 0, 0, 0)),
                pl.BlockSpec((192, 64), _const2),
                pl.BlockSpec((1, 64), _const2)],
      out_specs=pl.BlockSpec((3136, 64), lambda i: (i, 0)),
      scratch_shapes=[pltpu.VMEM((114, 114, 64), _BF),
                      pltpu.VMEM((12544, 192), _BF)],
      compiler_params=pltpu.CompilerParams(
          dimension_semantics=("parallel",),
          vmem_limit_bytes=_VMEM_LIMIT),
  )(colA, wm, bm)


def _head_body(x_ref, w_ref, b_ref, o_ref):
  m = x_ref.shape[0] // 49
  x = x_ref[...].astype(_F32).reshape(m, 49, 2048)
  feat = jnp.mean(x, axis=1).astype(_BF)
  o_ref[...] = jnp.dot(feat, w_ref[...], preferred_element_type=_F32) + b_ref[...]


def _head(x, fc_w, fc_b):
  n = x.shape[0] // 49
  classes = fc_w.shape[1]
  return pl.pallas_call(
      _head_body,
      out_shape=jax.ShapeDtypeStruct((n, classes), _F32),
      grid=(1,),
      in_specs=[pl.BlockSpec((n * 49, 2048), _const2),
                pl.BlockSpec((2048, classes), _const2),
                pl.BlockSpec((1, classes), _const2)],
      out_specs=pl.BlockSpec((n, classes), _const2),
      compiler_params=pltpu.CompilerParams(
          dimension_semantics=("arbitrary",),
          vmem_limit_bytes=_VMEM_LIMIT),
  )(x.astype(_BF), fc_w.astype(_BF), fc_b.reshape(1, classes).astype(_F32))


_STAGE_GEOM = {
    (0, 0): (56, 64, 64, 256, 1, 2),
    (0, 1): (56, 256, 64, 256, 1, 2),
    (1, 0): (56, 256, 128, 512, 2, 2),
    (1, 1): (28, 512, 128, 512, 1, 2),
    (2, 0): (28, 512, 256, 1024, 2, 2),
    (2, 1): (14, 1024, 256, 1024, 1, 4),
    (3, 0): (14, 1024, 512, 2048, 2, 8),
    (3, 1): (7, 2048, 512, 2048, 1, 8),
}


def kernel(stem_conv1_w, stem_conv1_bias, L0_B0_conv1_w, L0_B0_conv1_bias, L0_B0_conv2_w, L0_B0_conv2_bias, L0_B0_conv3_w, L0_B0_conv3_bias, L0_B0_downsample_w, L0_B0_downsample_bias, L0_B1_conv1_w, L0_B1_conv1_bias, L0_B1_conv2_w, L0_B1_conv2_bias, L0_B1_conv3_w, L0_B1_conv3_bias, L0_B2_conv1_w, L0_B2_conv1_bias, L0_B2_conv2_w, L0_B2_conv2_bias, L0_B2_conv3_w, L0_B2_conv3_bias, L1_B0_conv1_w, L1_B0_conv1_bias, L1_B0_conv2_w, L1_B0_conv2_bias, L1_B0_conv3_w, L1_B0_conv3_bias, L1_B0_downsample_w, L1_B0_downsample_bias, L1_B1_conv1_w, L1_B1_conv1_bias, L1_B1_conv2_w, L1_B1_conv2_bias, L1_B1_conv3_w, L1_B1_conv3_bias, L1_B2_conv1_w, L1_B2_conv1_bias, L1_B2_conv2_w, L1_B2_conv2_bias, L1_B2_conv3_w, L1_B2_conv3_bias, L1_B3_conv1_w, L1_B3_conv1_bias, L1_B3_conv2_w, L1_B3_conv2_bias, L1_B3_conv3_w, L1_B3_conv3_bias, L2_B0_conv1_w, L2_B0_conv1_bias, L2_B0_conv2_w, L2_B0_conv2_bias, L2_B0_conv3_w, L2_B0_conv3_bias, L2_B0_downsample_w, L2_B0_downsample_bias, L2_B1_conv1_w, L2_B1_conv1_bias, L2_B1_conv2_w, L2_B1_conv2_bias, L2_B1_conv3_w, L2_B1_conv3_bias, L2_B2_conv1_w, L2_B2_conv1_bias, L2_B2_conv2_w, L2_B2_conv2_bias, L2_B2_conv3_w, L2_B2_conv3_bias, L2_B3_conv1_w, L2_B3_conv1_bias, L2_B3_conv2_w, L2_B3_conv2_bias, L2_B3_conv3_w, L2_B3_conv3_bias, L2_B4_conv1_w, L2_B4_conv1_bias, L2_B4_conv2_w, L2_B4_conv2_bias, L2_B4_conv3_w, L2_B4_conv3_bias, L2_B5_conv1_w, L2_B5_conv1_bias, L2_B5_conv2_w, L2_B5_conv2_bias, L2_B5_conv3_w, L2_B5_conv3_bias, L3_B0_conv1_w, L3_B0_conv1_bias, L3_B0_conv2_w, L3_B0_conv2_bias, L3_B0_conv3_w, L3_B0_conv3_bias, L3_B0_downsample_w, L3_B0_downsample_bias, L3_B1_conv1_w, L3_B1_conv1_bias, L3_B1_conv2_w, L3_B1_conv2_bias, L3_B1_conv3_w, L3_B1_conv3_bias, L3_B2_conv1_w, L3_B2_conv1_bias, L3_B2_conv2_w, L3_B2_conv2_bias, L3_B2_conv3_w, L3_B2_conv3_bias, fc_w, fc_b, x):
  n = x.shape[0]
  args = locals()
  a = _stem(x, stem_conv1_w, stem_conv1_bias)

  nblocks = (3, 4, 6, 3)
  for li in range(4):
    for bi in range(nblocks[li]):
      pre = f"L{li}_B{bi}_"
      p1 = (args[pre + "conv1_w"], args[pre + "conv1_bias"])
      p2 = (args[pre + "conv2_w"], args[pre + "conv2_bias"])
      p3 = (args[pre + "conv3_w"], args[pre + "conv3_bias"])
      pd = None
      if bi == 0:
        pd = (args[pre + "downsample_w"], args[pre + "downsample_bias"])
      h, cin, width, cout, stride, g = _STAGE_GEOM[(li, min(bi, 1))]
      g = min(g, n)
      a = _bottleneck(a, p1, p2, p3, pd, n, h, h, cin, width, cout, stride, g)

  return _head(a, fc_w, fc_b)
